# baseline (speedup 1.0000x reference)
import numpy as np

import jax
import jax.numpy as jnp
from jax.experimental import pallas as pl
from jax.experimental.pallas import tpu as pltpu
from jax.experimental.shard_map import shard_map
from jax.sharding import Mesh, NamedSharding, PartitionSpec as P


def _router_block(x_ref, w1_ref, b1_ref, w2_ref, b2_ref, rw_ref, tw_ref, ti_ref):
    e_dim = rw_ref.shape[-1]
    h = jnp.dot(x_ref[...], w1_ref[...], preferred_element_type=jnp.float32)
    h = jnp.maximum(h + b1_ref[...], 0.0)
    logits = jnp.dot(h, w2_ref[...], preferred_element_type=jnp.float32)
    logits = logits + b2_ref[...]

    ids = jax.lax.broadcasted_iota(jnp.int32, logits.shape, 1)
    m1 = jnp.max(logits, axis=1, keepdims=True)
    i1 = jnp.min(jnp.where(logits == m1, ids, e_dim), axis=1, keepdims=True)
    masked = jnp.where(ids == i1, -jnp.inf, logits)
    m2 = jnp.max(masked, axis=1, keepdims=True)
    i2 = jnp.min(jnp.where(masked == m2, ids, e_dim), axis=1, keepdims=True)

    e = jnp.exp(logits - m1)
    z = jnp.sum(e, axis=1, keepdims=True)
    rw_ref[...] = e / z

    w1v = 1.0 / (1.0 + jnp.exp(m2 - m1))
    tw_ref[...] = jnp.concatenate([w1v, 1.0 - w1v], axis=1)
    ti_ref[...] = jnp.concatenate([i1, i2], axis=1)


def _router_shard(x, W1, b1, W2, b2):
    t, d = x.shape
    h_dim = W1.shape[1]
    e_dim = W2.shape[1]
    bt = min(512, t)

    rw, tw, ti = pl.pallas_call(
        _router_block,
        grid=(t // bt,),
        in_specs=[
            pl.BlockSpec((bt, d), lambda i: (i, 0)),
            pl.BlockSpec((d, h_dim), lambda i: (0, 0)),
            pl.BlockSpec((1, h_dim), lambda i: (0, 0)),
            pl.BlockSpec((h_dim, e_dim), lambda i: (0, 0)),
            pl.BlockSpec((1, e_dim), lambda i: (0, 0)),
        ],
        out_specs=[
            pl.BlockSpec((bt, e_dim), lambda i: (i, 0)),
            pl.BlockSpec((bt, 2), lambda i: (i, 0)),
            pl.BlockSpec((bt, 2), lambda i: (i, 0)),
        ],
        out_shape=[
            jax.ShapeDtypeStruct((t, e_dim), jnp.float32),
            jax.ShapeDtypeStruct((t, 2), jnp.float32),
            jax.ShapeDtypeStruct((t, 2), jnp.int32),
        ],
        compiler_params=pltpu.CompilerParams(
            dimension_semantics=("arbitrary",),
            vmem_limit_bytes=60 * 1024 * 1024,
        ),
    )(x, W1, b1.reshape(1, h_dim), W2, b2.reshape(1, e_dim))
    return rw, tw, ti


def kernel(x, W1, b1, W2, b2, inference_state):
    del inference_state
    t = x.shape[0]
    devs = jax.devices()
    if len(devs) >= 2 and t % (2 * 512) == 0:
        mesh = Mesh(np.array(devs[:2]), ("d",))
        repl = NamedSharding(mesh, P())
        xs = jax.device_put(x, NamedSharding(mesh, P("d", None)))
        w1s = jax.device_put(W1, repl)
        b1s = jax.device_put(b1, repl)
        w2s = jax.device_put(W2, repl)
        b2s = jax.device_put(b2, repl)
        rw, tw, ti = shard_map(
            _router_shard,
            mesh=mesh,
            in_specs=(P("d", None), P(None, None), P(None), P(None, None), P(None)),
            out_specs=(P("d", None), P("d", None), P("d", None)),
            check_rep=False,
        )(xs, w1s, b1s, w2s, b2s)
    else:
        rw, tw, ti = _router_shard(x, W1, b1, W2, b2)
    return (tw, rw, ti)


# speedup vs baseline: 2.1160x; 2.1160x over previous
import jax
import jax.numpy as jnp
from jax.experimental import pallas as pl
from jax.experimental.pallas import tpu as pltpu


def _router_block(x_ref, w1_ref, b1_ref, w2_ref, b2_ref, rw_ref, tw_ref, ti_ref):
    e_dim = rw_ref.shape[-1]
    h = jnp.dot(x_ref[...], w1_ref[...], preferred_element_type=jnp.float32)
    h = jnp.maximum(h + b1_ref[...], 0.0)
    logits = jnp.dot(h, w2_ref[...], preferred_element_type=jnp.float32)
    logits = logits + b2_ref[...]

    ids = jax.lax.broadcasted_iota(jnp.int32, logits.shape, 1)
    m1 = jnp.max(logits, axis=1, keepdims=True)
    i1 = jnp.min(jnp.where(logits == m1, ids, e_dim), axis=1, keepdims=True)
    masked = jnp.where(ids == i1, -jnp.inf, logits)
    m2 = jnp.max(masked, axis=1, keepdims=True)
    i2 = jnp.min(jnp.where(masked == m2, ids, e_dim), axis=1, keepdims=True)

    e = jnp.exp(logits - m1)
    z = jnp.sum(e, axis=1, keepdims=True)
    rw_ref[...] = e / z

    w1v = 1.0 / (1.0 + jnp.exp(m2 - m1))
    tw_ref[...] = jnp.concatenate([w1v, 1.0 - w1v], axis=1)
    ti_ref[...] = jnp.concatenate([i1, i2], axis=1)


def kernel(x, W1, b1, W2, b2, inference_state):
    del inference_state
    t, d = x.shape
    h_dim = W1.shape[1]
    e_dim = W2.shape[1]
    bt = min(512, t)

    rw, tw, ti = pl.pallas_call(
        _router_block,
        grid=(t // bt,),
        in_specs=[
            pl.BlockSpec((bt, d), lambda i: (i, 0)),
            pl.BlockSpec((d, h_dim), lambda i: (0, 0)),
            pl.BlockSpec((1, h_dim), lambda i: (0, 0)),
            pl.BlockSpec((h_dim, e_dim), lambda i: (0, 0)),
            pl.BlockSpec((1, e_dim), lambda i: (0, 0)),
        ],
        out_specs=[
            pl.BlockSpec((bt, e_dim), lambda i: (i, 0)),
            pl.BlockSpec((bt, 2), lambda i: (i, 0)),
            pl.BlockSpec((bt, 2), lambda i: (i, 0)),
        ],
        out_shape=[
            jax.ShapeDtypeStruct((t, e_dim), jnp.float32),
            jax.ShapeDtypeStruct((t, 2), jnp.float32),
            jax.ShapeDtypeStruct((t, 2), jnp.int32),
        ],
        compiler_params=pltpu.CompilerParams(
            dimension_semantics=("arbitrary",),
            vmem_limit_bytes=60 * 1024 * 1024,
        ),
    )(x, W1, b1.reshape(1, h_dim), W2, b2.reshape(1, e_dim))
    return (tw, rw, ti)
